# baseline (speedup 1.0000x reference)
import math

import jax
import jax.numpy as jnp
import numpy as np
from jax import lax
from jax.experimental import pallas as pl
from jax.experimental.pallas import tpu as pltpu
from jax.experimental.pallas import tpu_sc as plsc

MAXLEN = 512
NUM_CORES = 2
NUM_SUBCORES = 16
NW = NUM_CORES * NUM_SUBCORES
LANES = 16


def _make_pe_np(emb: int) -> np.ndarray:
    pe = np.zeros((MAXLEN, emb), dtype=np.float32)
    position = np.arange(0, MAXLEN, dtype=np.float32)[:, None]
    div_term = np.exp(
        np.arange(0, emb, 2, dtype=np.float32) * -(math.log(10000.0) / emb)
    )
    pe[:, 0::2] = np.sin(position * div_term)
    pe[:, 1::2] = np.cos(position * div_term)
    return pe


def kernel(input, weight):
    B, L = input.shape
    V, D = weight.shape
    factor = math.sqrt(D)
    total = B * L
    rpw = total // NW
    seq_per_w = B // NW
    S = 2
    C = S * L
    nch = seq_per_w // S

    pe = jnp.asarray(_make_pe_np(D)[:L])

    mesh = plsc.VectorSubcoreMesh(
        core_axis_name="c",
        subcore_axis_name="s",
        num_cores=NUM_CORES,
        num_subcores=NUM_SUBCORES,
    )

    @jax.jit
    def run(idx_flat, w, pe_arr):
        @pl.kernel(
            out_type=jax.ShapeDtypeStruct((total, D), jnp.float32),
            mesh=mesh,
            compiler_params=pltpu.CompilerParams(use_tc_tiling_on_sc=False),
            scratch_types=[
                pltpu.VMEM((rpw,), jnp.int32),
                pltpu.VMEM((L, D), jnp.float32),
                pltpu.VMEM((C, D), jnp.float32),
                pltpu.SemaphoreType.DMA,
            ],
        )
        def body(idx_hbm, w_hbm, pe_hbm, out_hbm, idx_all, pe_v, buf, gsem):
            wid = lax.axis_index("s") * NUM_CORES + lax.axis_index("c")
            base = wid * rpw
            pltpu.sync_copy(idx_hbm.at[pl.ds(base, rpw)], idx_all)
            pltpu.sync_copy(pe_hbm, pe_v)

            def chunk_body(g, _):
                off = g * C
                pltpu.async_copy(
                    w_hbm.at[idx_all.at[pl.ds(off, C)]], buf, gsem
                ).wait()

                def pos_body(p, _):
                    for j in range(D // LANES):
                        cs = pl.ds(j * LANES, LANES)
                        pev = pe_v[p, cs]
                        for s in range(S):
                            r = s * L + p
                            buf[r, cs] = buf[r, cs] * factor + pev
                    return 0

                lax.fori_loop(0, L, pos_body, 0)
                pltpu.sync_copy(buf, out_hbm.at[pl.ds(base + off, C)])
                return 0

            lax.fori_loop(0, nch, chunk_body, 0)

        return body(idx_flat, w, pe_arr)

    out_flat = run(input.reshape(total), weight, pe)
    return out_flat.reshape(B, L, D)
